# baseline (speedup 1.0000x reference)
import functools

import jax
import jax.numpy as jnp
from jax import lax
from jax.experimental import pallas as pl
from jax.experimental.pallas import tpu as pltpu
from jax.experimental.pallas import tpu_sc as plsc

_B = 16384
_F = 26
_K = 16
_TOTAL = _B * _F
_NW = 32
_PER_W = _TOTAL // _NW
_NCHUNK = 4
_CHUNK = _PER_W // _NCHUNK


def _gather_rows(idx_flat, table):
    mesh = plsc.VectorSubcoreMesh(core_axis_name="c", subcore_axis_name="s")

    @functools.partial(
        pl.kernel,
        mesh=mesh,
        out_type=jax.ShapeDtypeStruct((_TOTAL, _K), jnp.float32),
        scratch_types=[
            pltpu.VMEM((_PER_W,), jnp.int32),
            pltpu.VMEM((_CHUNK, _K), jnp.float32),
            pltpu.SemaphoreType.DMA,
        ],
        compiler_params=pltpu.CompilerParams(use_tc_tiling_on_sc=False),
    )
    def k(idx_hbm, table_hbm, out_hbm, idx_v, rows_v, gsem):
        wid = lax.axis_index("s") * 2 + lax.axis_index("c")
        base = wid * _PER_W
        pltpu.sync_copy(idx_hbm.at[pl.ds(base, _PER_W)], idx_v)
        for ci in range(_NCHUNK):
            pltpu.async_copy(
                table_hbm.at[idx_v.at[pl.ds(ci * _CHUNK, _CHUNK)]],
                rows_v,
                gsem,
            ).wait()
            pltpu.sync_copy(
                rows_v, out_hbm.at[pl.ds(base + ci * _CHUNK, _CHUNK)]
            )

    return k(idx_flat, table)


def kernel(idx, val, table):
    idx_flat = idx.reshape(_TOTAL).astype(jnp.int32)
    rows = _gather_rows(idx_flat, table)
    em_w = rows.reshape(_B, _F, _K)
    val_e = val[..., None]
    return (em_w, val_e)


# speedup vs baseline: 1.2967x; 1.2967x over previous
import functools

import jax
import jax.numpy as jnp
from jax import lax
from jax.experimental import pallas as pl
from jax.experimental.pallas import tpu as pltpu
from jax.experimental.pallas import tpu_sc as plsc

_B = 16384
_F = 26
_K = 16
_NW = 32
_BR = _B // _NW
_NCHUNK = 4
_RCH = _BR // _NCHUNK


def _em_pallas(idx, table):
    mesh = plsc.VectorSubcoreMesh(core_axis_name="c", subcore_axis_name="s")

    @functools.partial(
        pl.kernel,
        mesh=mesh,
        out_type=jax.ShapeDtypeStruct((_B, _F, _K), jnp.float32),
        scratch_types=[
            pltpu.VMEM((_BR, _F), jnp.int32),
            pltpu.VMEM((_RCH, _F, _K), jnp.float32),
            pltpu.SemaphoreType.DMA,
        ],
        compiler_params=pltpu.CompilerParams(use_tc_tiling_on_sc=False),
    )
    def k(idx_hbm, table_hbm, out_hbm, idx_v, rows_v, gsem):
        wid = lax.axis_index("s") * 2 + lax.axis_index("c")
        r0 = wid * _BR
        pltpu.sync_copy(idx_hbm.at[pl.ds(r0, _BR), :], idx_v)
        for ci in range(_NCHUNK):
            def fire(r, carry, ci=ci):
                pltpu.make_async_copy(
                    table_hbm.at[idx_v.at[ci * _RCH + r]],
                    rows_v.at[r],
                    gsem,
                ).start()
                return carry
            lax.fori_loop(0, _RCH, fire, 0)
            pltpu.make_async_copy(
                out_hbm.at[pl.ds(r0 + ci * _RCH, _RCH)], rows_v, gsem
            ).wait()
            pltpu.sync_copy(
                rows_v, out_hbm.at[pl.ds(r0 + ci * _RCH, _RCH)]
            )

    return k(idx, table)


def kernel(idx, val, table):
    if idx.dtype != jnp.int32:
        idx = idx.astype(jnp.int32)
    em_w = _em_pallas(idx, table)
    val_e = val[..., None]
    return (em_w, val_e)


# speedup vs baseline: 1.5618x; 1.2044x over previous
import functools

import jax
import jax.numpy as jnp
from jax import lax
from jax.experimental import pallas as pl
from jax.experimental.pallas import tpu as pltpu
from jax.experimental.pallas import tpu_sc as plsc

_B = 16384
_F = 26
_K = 16
_V = 1000000
_NW = 32
_BR = _B // _NW
_NLB = _BR // 128


def _em_pallas(idx_t, table):
    mesh = plsc.VectorSubcoreMesh(core_axis_name="c", subcore_axis_name="s")

    @functools.partial(
        pl.kernel,
        mesh=mesh,
        out_type=jax.ShapeDtypeStruct((_F, 2, _B // 128, 8, 128),
                                      jnp.float32),
        scratch_types=[
            pltpu.VMEM((_F, _BR), jnp.int32),
            pltpu.VMEM((_BR, _K), jnp.float32),
            pltpu.VMEM((_K, _BR), jnp.float32),
            pltpu.SemaphoreType.DMA,
        ],
        compiler_params=pltpu.CompilerParams(
            use_tc_tiling_on_sc=False, needs_layout_passes=False
        ),
    )
    def k(idxt_hbm, tab_hbm, out_hbm, idx_v, rows_v, t_v, gsem):
        wid = lax.axis_index("s") * 2 + lax.axis_index("c")
        c0 = wid * _BR
        pltpu.sync_copy(idxt_hbm.at[:, pl.ds(c0, _BR)], idx_v)

        def body(f, carry):
            pltpu.async_copy(
                tab_hbm.at[idx_v.at[f]], rows_v, gsem
            ).wait()

            def tr(c, carry2):
                rows16 = c * 16 + lax.iota(jnp.int32, 16)
                for kk in range(_K):
                    cols16 = jnp.full((16,), kk, jnp.int32)
                    t_v[kk, pl.ds(c * 16, 16)] = plsc.load_gather(
                        rows_v, [rows16, cols16]
                    )
                return carry2
            lax.fori_loop(0, _BR // 16, tr, 0)

            for kb in range(2):
                for lb in range(_NLB):
                    pltpu.sync_copy(
                        t_v.at[pl.ds(kb * 8, 8), pl.ds(lb * 128, 128)],
                        out_hbm.at[f, kb, wid * _NLB + lb],
                    )
            return carry

        lax.fori_loop(0, _F, body, 0)

    return k(idx_t, table)


def kernel(idx, val, table):
    if idx.dtype != jnp.int32:
        idx = idx.astype(jnp.int32)
    idx_t = idx.T
    out5d = _em_pallas(idx_t, table)
    em_w = (out5d.transpose(2, 4, 0, 1, 3)
            .reshape(_B, _F, _K))
    val_e = val[..., None]
    return (em_w, val_e)
